# Initial kernel scaffold; baseline (speedup 1.0000x reference)
import jax
import jax.numpy as jnp
from jax.experimental import pallas as pl


def kernel(x, edge_index, batch, w1, b1, w2, b2, Wg1, bg1, Wg2, bg2, Wc1, bc1, Wc2, bc2, Wm1, bm1, Wm2, bm2):
    raise NotImplementedError("write your pallas kernel here")


# speedup vs baseline: 1.4848x; 1.4848x over previous
import jax
import jax.numpy as jnp
from jax import lax
from jax.experimental import pallas as pl
from jax.experimental.pallas import tpu as pltpu

N = 50000
E = 800000
EN = E + N
C = 1000
NCH = EN // C
HID = 64
G = 512
PC = 2000
NPC = N // PC


def _deg_kernel(dst_ref, dis_ref):
    i = pl.program_id(0)

    @pl.when(i == 0)
    def _():
        dis_ref[...] = jnp.zeros_like(dis_ref)

    def body(j, carry):
        d = dst_ref[0, 0, j]
        dis_ref[pl.ds(d, 1), :] = dis_ref[pl.ds(d, 1), :] + 1.0
        return carry

    lax.fori_loop(0, C, body, 0)

    @pl.when(i == NCH - 1)
    def _():
        dis_ref[...] = lax.rsqrt(jnp.maximum(dis_ref[...], 1.0))


def _dense1_kernel(h_ref, w_ref, dis_ref, y_ref):
    y_ref[...] = jnp.dot(h_ref[...], w_ref[...],
                         preferred_element_type=jnp.float32) * dis_ref[...]


def _dense2_kernel(s_ref, dis_ref, bprev_ref, w_ref, y_ref):
    h = jnp.maximum(s_ref[...] * dis_ref[...] + bprev_ref[...], 0.0)
    y_ref[...] = jnp.dot(h, w_ref[...],
                         preferred_element_type=jnp.float32) * dis_ref[...]


def _scatter_kernel(y_ref, src_ref, dst_ref, out_ref):
    i = pl.program_id(0)

    @pl.when(i == 0)
    def _():
        out_ref[...] = jnp.zeros_like(out_ref)

    def body(j, carry):
        s = src_ref[0, 0, j]
        d = dst_ref[0, 0, j]
        out_ref[pl.ds(d, 1), :] = out_ref[pl.ds(d, 1), :] + y_ref[pl.ds(s, 1), :]
        return carry

    lax.fori_loop(0, C, body, 0)


def _pool_kernel(s_ref, dis_ref, bprev_ref, batch_ref,
                 wc1_ref, bc1_ref, wc2_ref, bc2_ref,
                 wm1_ref, bm1_ref, wm2_ref, bm2_ref,
                 logits_ref, mag_ref, gsum_ref, cnt_ref):
    i = pl.program_id(0)

    @pl.when(i == 0)
    def _():
        gsum_ref[...] = jnp.zeros_like(gsum_ref)
        cnt_ref[...] = jnp.zeros_like(cnt_ref)

    h = jnp.maximum(s_ref[...] * dis_ref[...] + bprev_ref[...], 0.0)
    bcol = batch_ref[0]
    onehot = (bcol == lax.broadcasted_iota(jnp.int32, (PC, G), 1)
              ).astype(jnp.float32)
    gsum_ref[...] = gsum_ref[...] + lax.dot_general(
        onehot, h, (((0,), (0,)), ((), ())),
        preferred_element_type=jnp.float32)
    cnt_ref[...] = cnt_ref[...] + lax.dot_general(
        onehot, jnp.ones((PC, 1), jnp.float32), (((0,), (0,)), ((), ())),
        preferred_element_type=jnp.float32)

    @pl.when(i == NPC - 1)
    def _():
        ge = gsum_ref[...] / jnp.maximum(cnt_ref[...], 1.0)
        hc = jnp.maximum(
            jnp.dot(ge, wc1_ref[...], preferred_element_type=jnp.float32)
            + bc1_ref[...], 0.0)
        logits_ref[...] = jnp.dot(
            hc, wc2_ref[...], preferred_element_type=jnp.float32) + bc2_ref[...]
        hm = jnp.maximum(
            jnp.dot(ge, wm1_ref[...], preferred_element_type=jnp.float32)
            + bm1_ref[...], 0.0)
        mag_ref[...] = jnp.dot(
            hm, wm2_ref[...], preferred_element_type=jnp.float32) + bm2_ref[...]


def _row_spec(width):
    return pl.BlockSpec((PC, width), lambda i: (i, 0))


def _const_spec(shape):
    return pl.BlockSpec(shape, lambda i: (0,) * len(shape))


def _edge_spec():
    return pl.BlockSpec((1, 1, C), lambda i: (i, 0, 0), memory_space=pltpu.SMEM)


def _scatter(y, src, dst):
    return pl.pallas_call(
        _scatter_kernel,
        grid=(NCH,),
        in_specs=[
            pl.BlockSpec((N, HID), lambda i: (0, 0)),
            _edge_spec(),
            _edge_spec(),
        ],
        out_specs=pl.BlockSpec((N, HID), lambda i: (0, 0)),
        out_shape=jax.ShapeDtypeStruct((N, HID), jnp.float32),
    )(y, src, dst)


def kernel(x, edge_index, batch, w1, b1, w2, b2, Wg1, bg1, Wg2, bg2,
           Wc1, bc1, Wc2, bc2, Wm1, bm1, Wm2, bm2):
    h = lax.conv_general_dilated(x, w1, (2,), [(3, 3)],
                                 dimension_numbers=('NCH', 'OIH', 'NCH'))
    h = jax.nn.relu(h + b1[None, :, None])
    h = lax.conv_general_dilated(h, w2, (2,), [(2, 2)],
                                 dimension_numbers=('NCH', 'OIH', 'NCH'))
    h = jax.nn.relu(h + b2[None, :, None])
    feat = jnp.mean(h, axis=2)

    loop = jnp.arange(N, dtype=edge_index.dtype)
    src = jnp.concatenate([edge_index[0], loop]).reshape(NCH, 1, C)
    dst = jnp.concatenate([edge_index[1], loop]).reshape(NCH, 1, C)

    dis = pl.pallas_call(
        _deg_kernel,
        grid=(NCH,),
        in_specs=[_edge_spec()],
        out_specs=pl.BlockSpec((N, 1), lambda i: (0, 0)),
        out_shape=jax.ShapeDtypeStruct((N, 1), jnp.float32),
    )(dst)

    y1 = pl.pallas_call(
        _dense1_kernel,
        grid=(NPC,),
        in_specs=[_row_spec(HID), _const_spec((HID, HID)), _row_spec(1)],
        out_specs=_row_spec(HID),
        out_shape=jax.ShapeDtypeStruct((N, HID), jnp.float32),
    )(feat, Wg1, dis)

    s1 = _scatter(y1, src, dst)

    y2 = pl.pallas_call(
        _dense2_kernel,
        grid=(NPC,),
        in_specs=[_row_spec(HID), _row_spec(1), _const_spec((1, HID)),
                  _const_spec((HID, HID))],
        out_specs=_row_spec(HID),
        out_shape=jax.ShapeDtypeStruct((N, HID), jnp.float32),
    )(s1, dis, bg1.reshape(1, HID), Wg2)

    s2 = _scatter(y2, src, dst)

    batch3 = batch.reshape(NPC, PC, 1)
    logits, mag = pl.pallas_call(
        _pool_kernel,
        grid=(NPC,),
        in_specs=[
            _row_spec(HID),
            _row_spec(1),
            _const_spec((1, HID)),
            pl.BlockSpec((1, PC, 1), lambda i: (i, 0, 0)),
            _const_spec((HID, HID // 2)),
            _const_spec((1, HID // 2)),
            _const_spec((HID // 2, 2)),
            _const_spec((1, 2)),
            _const_spec((HID, HID // 2)),
            _const_spec((1, HID // 2)),
            _const_spec((HID // 2, 1)),
            _const_spec((1, 1)),
        ],
        out_specs=[
            pl.BlockSpec((G, 2), lambda i: (0, 0)),
            pl.BlockSpec((G, 1), lambda i: (0, 0)),
        ],
        out_shape=[
            jax.ShapeDtypeStruct((G, 2), jnp.float32),
            jax.ShapeDtypeStruct((G, 1), jnp.float32),
        ],
        scratch_shapes=[pltpu.VMEM((G, HID), jnp.float32),
                        pltpu.VMEM((G, 1), jnp.float32)],
    )(s2, dis, bg2.reshape(1, HID), batch3, Wc1, bc1.reshape(1, -1),
      Wc2, bc2.reshape(1, -1), Wm1, bm1.reshape(1, -1), Wm2, bm2.reshape(1, -1))
    return (logits, mag)


# speedup vs baseline: 2.7190x; 1.8312x over previous
import jax
import jax.numpy as jnp
from jax import lax
from jax.experimental import pallas as pl
from jax.experimental.pallas import tpu as pltpu

N = 50000
E = 800000
EN = E + N
C = 1000
NCH = EN // C
HID = 64
G = 512
PC = 2000
NPC = N // PC


def _deg_kernel(dst_ref, dis_ref):
    i = pl.program_id(0)

    @pl.when(i == 0)
    def _():
        dis_ref[...] = jnp.zeros_like(dis_ref)

    def body(j, carry):
        d = dst_ref[0, 0, j]
        dis_ref[pl.ds(d, 1), :] = dis_ref[pl.ds(d, 1), :] + 1.0
        return carry

    lax.fori_loop(0, C, body, 0, unroll=8)

    @pl.when(i == NCH - 1)
    def _():
        dis_ref[...] = lax.rsqrt(jnp.maximum(dis_ref[...], 1.0))


def _dense1_kernel(h_ref, w_ref, dis_ref, y_ref):
    y_ref[...] = jnp.dot(h_ref[...], w_ref[...],
                         preferred_element_type=jnp.float32) * dis_ref[...]


def _dense2_kernel(s_ref, dis_ref, bprev_ref, w_ref, y_ref):
    h = jnp.maximum(s_ref[...] * dis_ref[...] + bprev_ref[...], 0.0)
    y_ref[...] = jnp.dot(h, w_ref[...],
                         preferred_element_type=jnp.float32) * dis_ref[...]


def _scatter_kernel(y_ref, src_ref, dst_ref, out_ref):
    i = pl.program_id(0)

    @pl.when(i == 0)
    def _():
        out_ref[...] = jnp.zeros_like(out_ref)

    def body(j, carry):
        s = src_ref[0, 0, j]
        d = dst_ref[0, 0, j]
        out_ref[pl.ds(d, 1), :] = out_ref[pl.ds(d, 1), :] + y_ref[pl.ds(s, 1), :]
        return carry

    lax.fori_loop(0, C, body, 0, unroll=8)


def _pool_kernel(s_ref, dis_ref, bprev_ref, batch_ref,
                 wc1_ref, bc1_ref, wc2_ref, bc2_ref,
                 wm1_ref, bm1_ref, wm2_ref, bm2_ref,
                 logits_ref, mag_ref, gsum_ref, cnt_ref):
    i = pl.program_id(0)

    @pl.when(i == 0)
    def _():
        gsum_ref[...] = jnp.zeros_like(gsum_ref)
        cnt_ref[...] = jnp.zeros_like(cnt_ref)

    h = jnp.maximum(s_ref[...] * dis_ref[...] + bprev_ref[...], 0.0)
    bcol = batch_ref[0]
    onehot = (bcol == lax.broadcasted_iota(jnp.int32, (PC, G), 1)
              ).astype(jnp.float32)
    gsum_ref[...] = gsum_ref[...] + lax.dot_general(
        onehot, h, (((0,), (0,)), ((), ())),
        preferred_element_type=jnp.float32)
    cnt_ref[...] = cnt_ref[...] + lax.dot_general(
        onehot, jnp.ones((PC, 1), jnp.float32), (((0,), (0,)), ((), ())),
        preferred_element_type=jnp.float32)

    @pl.when(i == NPC - 1)
    def _():
        ge = gsum_ref[...] / jnp.maximum(cnt_ref[...], 1.0)
        hc = jnp.maximum(
            jnp.dot(ge, wc1_ref[...], preferred_element_type=jnp.float32)
            + bc1_ref[...], 0.0)
        logits_ref[...] = jnp.dot(
            hc, wc2_ref[...], preferred_element_type=jnp.float32) + bc2_ref[...]
        hm = jnp.maximum(
            jnp.dot(ge, wm1_ref[...], preferred_element_type=jnp.float32)
            + bm1_ref[...], 0.0)
        mag_ref[...] = jnp.dot(
            hm, wm2_ref[...], preferred_element_type=jnp.float32) + bm2_ref[...]


def _row_spec(width):
    return pl.BlockSpec((PC, width), lambda i: (i, 0))


def _const_spec(shape):
    return pl.BlockSpec(shape, lambda i: (0,) * len(shape))


def _edge_spec():
    return pl.BlockSpec((1, 1, C), lambda i: (i, 0, 0), memory_space=pltpu.SMEM)


def _scatter(y, src, dst):
    return pl.pallas_call(
        _scatter_kernel,
        grid=(NCH,),
        in_specs=[
            pl.BlockSpec((N, HID), lambda i: (0, 0)),
            _edge_spec(),
            _edge_spec(),
        ],
        out_specs=pl.BlockSpec((N, HID), lambda i: (0, 0)),
        out_shape=jax.ShapeDtypeStruct((N, HID), jnp.float32),
    )(y, src, dst)


def kernel(x, edge_index, batch, w1, b1, w2, b2, Wg1, bg1, Wg2, bg2,
           Wc1, bc1, Wc2, bc2, Wm1, bm1, Wm2, bm2):
    h = lax.conv_general_dilated(x, w1, (2,), [(3, 3)],
                                 dimension_numbers=('NCH', 'OIH', 'NCH'))
    h = jax.nn.relu(h + b1[None, :, None])
    h = lax.conv_general_dilated(h, w2, (2,), [(2, 2)],
                                 dimension_numbers=('NCH', 'OIH', 'NCH'))
    h = jax.nn.relu(h + b2[None, :, None])
    feat = jnp.mean(h, axis=2)

    loop = jnp.arange(N, dtype=edge_index.dtype)
    src = jnp.concatenate([edge_index[0], loop]).reshape(NCH, 1, C)
    dst = jnp.concatenate([edge_index[1], loop]).reshape(NCH, 1, C)

    dis = pl.pallas_call(
        _deg_kernel,
        grid=(NCH,),
        in_specs=[_edge_spec()],
        out_specs=pl.BlockSpec((N, 1), lambda i: (0, 0)),
        out_shape=jax.ShapeDtypeStruct((N, 1), jnp.float32),
    )(dst)

    y1 = pl.pallas_call(
        _dense1_kernel,
        grid=(NPC,),
        in_specs=[_row_spec(HID), _const_spec((HID, HID)), _row_spec(1)],
        out_specs=_row_spec(HID),
        out_shape=jax.ShapeDtypeStruct((N, HID), jnp.float32),
    )(feat, Wg1, dis)

    s1 = _scatter(y1, src, dst)

    y2 = pl.pallas_call(
        _dense2_kernel,
        grid=(NPC,),
        in_specs=[_row_spec(HID), _row_spec(1), _const_spec((1, HID)),
                  _const_spec((HID, HID))],
        out_specs=_row_spec(HID),
        out_shape=jax.ShapeDtypeStruct((N, HID), jnp.float32),
    )(s1, dis, bg1.reshape(1, HID), Wg2)

    s2 = _scatter(y2, src, dst)

    batch3 = batch.reshape(NPC, PC, 1)
    logits, mag = pl.pallas_call(
        _pool_kernel,
        grid=(NPC,),
        in_specs=[
            _row_spec(HID),
            _row_spec(1),
            _const_spec((1, HID)),
            pl.BlockSpec((1, PC, 1), lambda i: (i, 0, 0)),
            _const_spec((HID, HID // 2)),
            _const_spec((1, HID // 2)),
            _const_spec((HID // 2, 2)),
            _const_spec((1, 2)),
            _const_spec((HID, HID // 2)),
            _const_spec((1, HID // 2)),
            _const_spec((HID // 2, 1)),
            _const_spec((1, 1)),
        ],
        out_specs=[
            pl.BlockSpec((G, 2), lambda i: (0, 0)),
            pl.BlockSpec((G, 1), lambda i: (0, 0)),
        ],
        out_shape=[
            jax.ShapeDtypeStruct((G, 2), jnp.float32),
            jax.ShapeDtypeStruct((G, 1), jnp.float32),
        ],
        scratch_shapes=[pltpu.VMEM((G, HID), jnp.float32),
                        pltpu.VMEM((G, 1), jnp.float32)],
    )(s2, dis, bg2.reshape(1, HID), batch3, Wc1, bc1.reshape(1, -1),
      Wc2, bc2.reshape(1, -1), Wm1, bm1.reshape(1, -1), Wm2, bm2.reshape(1, -1))
    return (logits, mag)
